# baseline (speedup 1.0000x reference)
import functools

import jax
import jax.numpy as jnp
from jax import lax
from jax.experimental import pallas as pl
from jax.experimental.pallas import tpu as pltpu
from jax.experimental.pallas import tpu_sc as plsc

N = 50000
E = 800000
NS = 16
SH = 4
OUT = 28
DEB = 32
EATT = 4

ROW = 32
NC = 2
NSUB = 16
NW = NC * NSUB
SUB = 128
KSUB = 8
FLIGHT = SUB * KSUB
FPW = -(-E // (NW * FLIGHT))
E_PAD = NW * FPW * FLIGHT
NCHUNK = E_PAD // SUB

TBN = 2000
AEB = 4096

ACC_PER_TILE = 3136
ACC = NSUB * ACC_PER_TILE
HALF = 16
FPT = E_PAD // (FLIGHT * NSUB)
BE = 4096
BE4 = BE // 4
BN = 400


def _gather_body(table_hbm, idx2_hbm, gout_hbm, idxs, idxd, rows_s, rows_d,
                 sem_s, sem_d):
    wid = lax.axis_index("s") * NC + lax.axis_index("c")

    def flight(f, carry):
        c0 = pl.multiple_of((wid * FPW + f) * KSUB, KSUB)
        e0 = pl.multiple_of(c0 * SUB, FLIGHT)
        pltpu.sync_copy(idx2_hbm.at[0, pl.ds(c0, KSUB)], idxs)
        pltpu.sync_copy(idx2_hbm.at[1, pl.ds(c0, KSUB)], idxd)
        cps = [
            pltpu.async_copy(table_hbm.at[idxs.at[j]],
                             rows_s.at[pl.ds(j * SUB, SUB)], sem_s)
            for j in range(KSUB)
        ]
        cpd = [
            pltpu.async_copy(table_hbm.at[idxd.at[j]],
                             rows_d.at[pl.ds(j * SUB, SUB)], sem_d)
            for j in range(KSUB)
        ]
        for cp in cps:
            cp.wait()
        for cp in cpd:
            cp.wait()
        pltpu.sync_copy(rows_s, gout_hbm.at[0, pl.ds(e0, FLIGHT)])
        pltpu.sync_copy(rows_d, gout_hbm.at[1, pl.ds(e0, FLIGHT)])
        return carry

    lax.fori_loop(0, FPW, flight, 0)


def _scatter_body(dst_hbm, msg_hbm, zeros_hbm, out_hbm, idxd, rows, acc, sem):
    c = lax.axis_index("c")
    s = lax.axis_index("s")
    pltpu.sync_copy(zeros_hbm, rows)
    base = pl.multiple_of(s * ACC_PER_TILE, 8)
    for off in (0, FLIGHT, 2 * FLIGHT):
        pltpu.sync_copy(rows, acc.at[pl.ds(base + off, FLIGHT)])
    pltpu.sync_copy(rows.at[pl.ds(0, ACC_PER_TILE - 3 * FLIGHT)],
                    acc.at[pl.ds(base + 3 * FLIGHT, ACC_PER_TILE - 3 * FLIGHT)])
    plsc.subcore_barrier()

    def flight(f, carry):
        gf = s * FPT + f
        c0 = pl.multiple_of(gf * KSUB, KSUB)
        r0 = pl.multiple_of(gf * (FLIGHT // 4), FLIGHT // 4)
        pltpu.sync_copy(dst_hbm.at[pl.ds(c0, KSUB)], idxd)
        for k in range(4):
            pltpu.sync_copy(
                msg_hbm.at[pl.ds(r0, FLIGHT // 4),
                           pl.ds(k * ROW + c * HALF, HALF)],
                rows.at[pl.ds(k * (FLIGHT // 4), FLIGHT // 4)])
        cps = [
            pltpu.async_copy(rows.at[pl.ds(j * SUB, SUB)],
                             acc.at[idxd.at[j]], sem, add=True)
            for j in range(KSUB)
        ]
        for cp in cps:
            cp.wait()
        return carry

    lax.fori_loop(0, FPT, flight, 0)
    plsc.subcore_barrier()
    pltpu.sync_copy(acc.at[pl.ds(base, ACC_PER_TILE)],
                    out_hbm.at[c, pl.ds(base, ACC_PER_TILE)])


@functools.lru_cache(maxsize=None)
def _sc_kernels():
    mesh = plsc.VectorSubcoreMesh(core_axis_name="c", subcore_axis_name="s",
                                  num_cores=NC, num_subcores=NSUB)
    sc_params = pltpu.CompilerParams(use_tc_tiling_on_sc=False)
    gather = pl.kernel(
        _gather_body,
        compiler_params=sc_params,
        out_type=jax.ShapeDtypeStruct((2, E_PAD, ROW), jnp.float32),
        mesh=mesh,
        scratch_types=[
            pltpu.VMEM((KSUB, SUB), jnp.int32),
            pltpu.VMEM((KSUB, SUB), jnp.int32),
            pltpu.VMEM((FLIGHT, ROW), jnp.float32),
            pltpu.VMEM((FLIGHT, ROW), jnp.float32),
            pltpu.SemaphoreType.DMA,
            pltpu.SemaphoreType.DMA,
        ],
    )
    scatter = pl.kernel(
        _scatter_body,
        compiler_params=sc_params,
        out_type=jax.ShapeDtypeStruct((NC, ACC, HALF), jnp.float32),
        mesh=mesh,
        scratch_types=[
            pltpu.VMEM((KSUB, SUB), jnp.int32),
            pltpu.VMEM((FLIGHT, HALF), jnp.float32),
            pltpu.VMEM_SHARED((ACC, HALF), jnp.float32),
            pltpu.SemaphoreType.DMA,
        ],
    )
    return gather, scatter


def _edge_body(gs_ref, gd_ref, ea_ref, S_ref, Px_ref, Py_ref, Pz_ref,
               A_ref, B_ref, b1_ref, C_ref, b2_ref,
               Da_ref, Ea_ref, Fa_ref, Db_ref, Eb_ref, Fb_ref, bt1_ref,
               G_ref, bt2_ref, Wo0_ref, Wox_ref, Woy_ref, Woz_ref, bo_ref,
               msg_ref):
    f32 = jnp.float32

    def dot(a, b):
        return jnp.dot(a, b, preferred_element_type=f32)

    gs = gs_ref[0]
    gd = gd_ref[0]
    q = gd - gs
    r2 = dot(q * q, S_ref[...])
    r = jnp.sqrt(r2 + 1e-8)
    rinv = 1.0 / r
    step = f32(5.0 / (DEB - 1))
    coeff = f32(-0.5) / (step * step)
    lane = lax.broadcasted_iota(jnp.int32, (1, 4 * ROW), 1) & (ROW - 1)
    offs = lane.astype(f32) * step
    demb = jnp.exp(coeff * (r - offs) ** 2)
    h1 = jnp.maximum(
        dot(ea_ref[...], A_ref[...]) + dot(demb, B_ref[...]) + b1_ref[...],
        0.0)
    eemb = dot(h1, C_ref[...]) + b2_ref[...]
    h2a = jnp.maximum(
        dot(eemb, Da_ref[...]) + dot(gs, Ea_ref[...]) + dot(gd, Fa_ref[...])
        + bt1_ref[...], 0.0)
    h2b = jnp.maximum(
        dot(eemb, Db_ref[...]) + dot(gs, Eb_ref[...]) + dot(gd, Fb_ref[...])
        + bt1_ref[...], 0.0)
    w4 = jnp.concatenate([dot(h2a, G_ref[...]), dot(h2b, G_ref[...])],
                         axis=1) + bt2_ref[...]
    u = w4 * gs
    v = u * rinv
    mx = v * dot(q, Px_ref[...])
    my = v * dot(q, Py_ref[...])
    mz = v * dot(q, Pz_ref[...])
    msg = (dot(u, Wo0_ref[...]) + dot(mx, Wox_ref[...]) +
           dot(my, Woy_ref[...]) + dot(mz, Woz_ref[...]) + bo_ref[...])
    msg_ref[...] = msg * f32(0.25)


def _table_body(x_ref, pos_ref, t_ref):
    t_ref[...] = jnp.concatenate(
        [x_ref[...], pos_ref[...],
         jnp.zeros((TBN, ROW - NS - 3), jnp.float32)], axis=1)


def _combine_body(p_ref, x_ref, o_ref):
    o_ref[...] = jnp.concatenate(
        [p_ref[0] + x_ref[...], p_ref[1][:, 0:OUT - HALF]], axis=1)


def kernel(x, pos, edge_attr, We1, be1, We2, be2, Wt1, bt1, Wt2, bt2, Wo, bo,
           edge_index):
    f32 = jnp.float32
    i32 = jnp.int32
    eye4 = jnp.eye(4, dtype=f32)
    kron4 = lambda w: jnp.kron(eye4, w)
    tile4 = lambda b: jnp.tile(b, 4).reshape(1, -1)

    pad = E_PAD - E
    apad = jnp.arange(pad, dtype=i32)
    src_p = jnp.concatenate([edge_index[0], apad % N]).reshape(NCHUNK, SUB)
    dst_flat = jnp.concatenate([edge_index[1], N + apad % (ACC - N)])
    dst_p = dst_flat.reshape(NCHUNK, SUB)
    idx2 = jnp.stack([src_p, dst_p])
    dst_ps = dst_flat.reshape(E_PAD // FLIGHT, FLIGHT // 4,
                              4).transpose(0, 2, 1).reshape(NCHUNK, SUB)

    table = pl.pallas_call(
        _table_body,
        grid=(N // TBN,),
        in_specs=[
            pl.BlockSpec((TBN, NS), lambda i: (i, 0)),
            pl.BlockSpec((TBN, 3), lambda i: (i, 0)),
        ],
        out_specs=pl.BlockSpec((TBN, ROW), lambda i: (i, 0)),
        out_shape=jax.ShapeDtypeStruct((N, ROW), f32),
    )(x, pos)
    ea4 = jnp.pad(edge_attr.reshape(E // 4, 4 * EATT),
                  ((0, (E_PAD - E) // 4), (0, 0)))

    sel = jnp.zeros((ROW, ROW), f32)
    S4 = kron4(sel.at[NS:NS + 3, :].set(1.0))
    Px4 = kron4(sel.at[NS, :].set(1.0))
    Py4 = kron4(sel.at[NS + 1, :].set(1.0))
    Pz4 = kron4(sel.at[NS + 2, :].set(1.0))

    A4 = kron4(We1[:EATT])
    B4 = kron4(We1[EATT:])
    C4 = kron4(We2)
    b1_4 = tile4(be1)
    b2_4 = tile4(be2)

    D = Wt1[:NS]
    E32 = jnp.pad(Wt1[NS:2 * NS], ((0, NS), (0, 0)))
    F32 = jnp.pad(Wt1[2 * NS:], ((0, NS), (0, 0)))
    sel01 = jnp.zeros((4, 2), f32).at[0, 0].set(1.0).at[1, 1].set(1.0)
    sel23 = jnp.zeros((4, 2), f32).at[2, 0].set(1.0).at[3, 1].set(1.0)
    Da = jnp.kron(sel01, D)
    Ea = jnp.kron(sel01, E32)
    Fa = jnp.kron(sel01, F32)
    Db = jnp.kron(sel23, D)
    Eb = jnp.kron(sel23, E32)
    Fb = jnp.kron(sel23, F32)
    bt1_2 = jnp.tile(bt1, 2).reshape(1, -1)
    G = jnp.kron(jnp.eye(2, dtype=f32),
                 jnp.pad(Wt2, ((0, 0), (0, NS))))
    bt2_4 = tile4(jnp.pad(bt2, (0, NS)))

    Wo_km = jnp.pad(
        Wo.reshape(NS, SH, OUT).transpose(1, 0, 2).reshape(NS * SH, OUT),
        ((0, 0), (0, ROW - OUT)))
    wo32 = lambda k: kron4(jnp.pad(Wo_km[k * NS:(k + 1) * NS],
                                   ((0, NS), (0, 0))))
    Wo0_4, Wox_4, Woy_4, Woz_4 = wo32(0), wo32(1), wo32(2), wo32(3)
    bo_4 = tile4(jnp.pad(bo, (0, ROW - OUT)))

    gather_kernel, scatter_kernel = _sc_kernels()
    gout = gather_kernel(table, idx2)
    g4 = gout.reshape(2, E_PAD // 4, 4 * ROW)

    grid_e = E_PAD // BE
    eb = lambda i: (i, 0)
    wb = lambda i: (0, 0)
    wspec = lambda shp: pl.BlockSpec(shp, wb)
    msg4 = pl.pallas_call(
        _edge_body,
        grid=(grid_e,),
        in_specs=[
            pl.BlockSpec((1, BE4, 4 * ROW), lambda i: (0, i, 0)),
            pl.BlockSpec((1, BE4, 4 * ROW), lambda i: (1, i, 0)),
            pl.BlockSpec((BE4, 4 * EATT), eb),
            wspec((128, 128)),
            wspec((128, 128)),
            wspec((128, 128)),
            wspec((128, 128)),
            wspec((4 * EATT, 64)),
            wspec((128, 64)),
            wspec((1, 64)),
            wspec((64, 64)),
            wspec((1, 64)),
            wspec((64, 96)),
            wspec((128, 96)),
            wspec((128, 96)),
            wspec((64, 96)),
            wspec((128, 96)),
            wspec((128, 96)),
            wspec((1, 96)),
            wspec((96, 64)),
            wspec((1, 128)),
            wspec((128, 128)),
            wspec((128, 128)),
            wspec((128, 128)),
            wspec((128, 128)),
            wspec((1, 128)),
        ],
        out_specs=pl.BlockSpec((BE4, 4 * ROW), eb),
        out_shape=jax.ShapeDtypeStruct((E_PAD // 4, 4 * ROW), f32),
    )(g4, g4, ea4, S4, Px4, Py4, Pz4, A4, B4, b1_4, C4, b2_4,
      Da, Ea, Fa, Db, Eb, Fb, bt1_2, G, bt2_4,
      Wo0_4, Wox_4, Woy_4, Woz_4, bo_4)

    zeros = jnp.zeros((FLIGHT, HALF), f32)
    partials = scatter_kernel(dst_ps, msg4, zeros)

    grid_n = N // BN
    out = pl.pallas_call(
        _combine_body,
        grid=(grid_n,),
        in_specs=[
            pl.BlockSpec((NC, BN, HALF), lambda i: (0, i, 0)),
            pl.BlockSpec((BN, NS), lambda i: (i, 0)),
        ],
        out_specs=pl.BlockSpec((BN, OUT), lambda i: (i, 0)),
        out_shape=jax.ShapeDtypeStruct((N, OUT), f32),
    )(partials[:, :N, :], x)
    return out


# speedup vs baseline: 1.0644x; 1.0644x over previous
import functools

import jax
import jax.numpy as jnp
from jax import lax
from jax.experimental import pallas as pl
from jax.experimental.pallas import tpu as pltpu
from jax.experimental.pallas import tpu_sc as plsc

N = 50000
E = 800000
NS = 16
SH = 4
OUT = 28
DEB = 32
EATT = 4

ROW = 32
NC = 2
NSUB = 16
NW = NC * NSUB
SUB = 128
KSUB = 8
FLIGHT = SUB * KSUB
FPW = -(-E // (NW * FLIGHT))
E_PAD = NW * FPW * FLIGHT
NCHUNK = E_PAD // SUB

TBN = 2000
AEB = 4096

ACC_PER_TILE = 3136
ACC = NSUB * ACC_PER_TILE
HALF = 16
FPT = E_PAD // (FLIGHT * NSUB)
BE = 4096
BE4 = BE // 4
BN = 400


def _gather_body(table_hbm, idx2_hbm, gout_hbm, idxs, idxd, rows_s, rows_d,
                 sem_s, sem_d):
    wid = lax.axis_index("s") * NC + lax.axis_index("c")

    def flight(f, carry):
        c0 = pl.multiple_of((wid * FPW + f) * KSUB, KSUB)
        e0 = pl.multiple_of(c0 * SUB, FLIGHT)
        pltpu.sync_copy(idx2_hbm.at[0, pl.ds(c0, KSUB)], idxs)
        pltpu.sync_copy(idx2_hbm.at[1, pl.ds(c0, KSUB)], idxd)
        cps = [
            pltpu.async_copy(table_hbm.at[idxs.at[j]],
                             rows_s.at[pl.ds(j * SUB, SUB)], sem_s)
            for j in range(KSUB)
        ]
        cpd = [
            pltpu.async_copy(table_hbm.at[idxd.at[j]],
                             rows_d.at[pl.ds(j * SUB, SUB)], sem_d)
            for j in range(KSUB)
        ]
        for cp in cps:
            cp.wait()
        for cp in cpd:
            cp.wait()
        pltpu.sync_copy(rows_s, gout_hbm.at[0, pl.ds(e0, FLIGHT)])
        pltpu.sync_copy(rows_d, gout_hbm.at[1, pl.ds(e0, FLIGHT)])
        return carry

    lax.fori_loop(0, FPW, flight, 0)


def _scatter_body(dst_hbm, msg_hbm, zeros_hbm, out_hbm, idxd, rows, acc, sem):
    c = lax.axis_index("c")
    s = lax.axis_index("s")
    pltpu.sync_copy(zeros_hbm, rows)
    base = pl.multiple_of(s * ACC_PER_TILE, 8)
    for off in (0, FLIGHT, 2 * FLIGHT):
        pltpu.sync_copy(rows, acc.at[pl.ds(base + off, FLIGHT)])
    pltpu.sync_copy(rows.at[pl.ds(0, ACC_PER_TILE - 3 * FLIGHT)],
                    acc.at[pl.ds(base + 3 * FLIGHT, ACC_PER_TILE - 3 * FLIGHT)])
    plsc.subcore_barrier()

    def flight(f, carry):
        gf = s * FPT + f
        c0 = pl.multiple_of(gf * KSUB, KSUB)
        e0 = pl.multiple_of(c0 * SUB, FLIGHT)
        pltpu.sync_copy(dst_hbm.at[pl.ds(c0, KSUB)], idxd)
        pltpu.sync_copy(msg_hbm.at[pl.ds(e0, FLIGHT), pl.ds(c * HALF, HALF)],
                        rows)
        cps = [
            pltpu.async_copy(rows.at[pl.ds(j * SUB, SUB)],
                             acc.at[idxd.at[j]], sem, add=True)
            for j in range(KSUB)
        ]
        for cp in cps:
            cp.wait()
        return carry

    lax.fori_loop(0, FPT, flight, 0)
    plsc.subcore_barrier()
    pltpu.sync_copy(acc.at[pl.ds(base, ACC_PER_TILE)],
                    out_hbm.at[c, pl.ds(base, ACC_PER_TILE)])


@functools.lru_cache(maxsize=None)
def _sc_kernels():
    mesh = plsc.VectorSubcoreMesh(core_axis_name="c", subcore_axis_name="s",
                                  num_cores=NC, num_subcores=NSUB)
    sc_params = pltpu.CompilerParams(use_tc_tiling_on_sc=False)
    gather = pl.kernel(
        _gather_body,
        compiler_params=sc_params,
        out_type=jax.ShapeDtypeStruct((2, E_PAD, ROW), jnp.float32),
        mesh=mesh,
        scratch_types=[
            pltpu.VMEM((KSUB, SUB), jnp.int32),
            pltpu.VMEM((KSUB, SUB), jnp.int32),
            pltpu.VMEM((FLIGHT, ROW), jnp.float32),
            pltpu.VMEM((FLIGHT, ROW), jnp.float32),
            pltpu.SemaphoreType.DMA,
            pltpu.SemaphoreType.DMA,
        ],
    )
    scatter = pl.kernel(
        _scatter_body,
        compiler_params=sc_params,
        out_type=jax.ShapeDtypeStruct((NC, ACC, HALF), jnp.float32),
        mesh=mesh,
        scratch_types=[
            pltpu.VMEM((KSUB, SUB), jnp.int32),
            pltpu.VMEM((FLIGHT, HALF), jnp.float32),
            pltpu.VMEM_SHARED((ACC, HALF), jnp.float32),
            pltpu.SemaphoreType.DMA,
        ],
    )
    return gather, scatter


def _edge_body(gs_ref, gd_ref, ea_ref, S_ref, Px_ref, Py_ref, Pz_ref,
               A_ref, B_ref, b1_ref, C_ref, b2_ref,
               Da_ref, Ea_ref, Fa_ref, Db_ref, Eb_ref, Fb_ref, bt1_ref,
               G_ref, bt2_ref, Wo0_ref, Wox_ref, Woy_ref, Woz_ref, bo_ref,
               msg_ref):
    f32 = jnp.float32

    def dot(a, b):
        return jnp.dot(a, b, preferred_element_type=f32)

    gs = gs_ref[0]
    gd = gd_ref[0]
    q = gd - gs
    r2 = dot(q * q, S_ref[...])
    r = jnp.sqrt(r2 + 1e-8)
    rinv = 1.0 / r
    step = f32(5.0 / (DEB - 1))
    coeff = f32(-0.5) / (step * step)
    lane = lax.broadcasted_iota(jnp.int32, (1, 4 * ROW), 1) & (ROW - 1)
    offs = lane.astype(f32) * step
    demb = jnp.exp(coeff * (r - offs) ** 2)
    h1 = jnp.maximum(
        dot(ea_ref[...], A_ref[...]) + dot(demb, B_ref[...]) + b1_ref[...],
        0.0)
    eemb = dot(h1, C_ref[...]) + b2_ref[...]
    h2a = jnp.maximum(
        dot(eemb, Da_ref[...]) + dot(gs, Ea_ref[...]) + dot(gd, Fa_ref[...])
        + bt1_ref[...], 0.0)
    h2b = jnp.maximum(
        dot(eemb, Db_ref[...]) + dot(gs, Eb_ref[...]) + dot(gd, Fb_ref[...])
        + bt1_ref[...], 0.0)
    w4 = jnp.concatenate([dot(h2a, G_ref[...]), dot(h2b, G_ref[...])],
                         axis=1) + bt2_ref[...]
    u = w4 * gs
    v = u * rinv
    mx = v * dot(q, Px_ref[...])
    my = v * dot(q, Py_ref[...])
    mz = v * dot(q, Pz_ref[...])
    msg = (dot(u, Wo0_ref[...]) + dot(mx, Wox_ref[...]) +
           dot(my, Woy_ref[...]) + dot(mz, Woz_ref[...]) + bo_ref[...])
    msg_ref[...] = msg * f32(0.25)


def _table_body(x_ref, pos_ref, t_ref):
    t_ref[...] = jnp.concatenate(
        [x_ref[...], pos_ref[...],
         jnp.zeros((TBN, ROW - NS - 3), jnp.float32)], axis=1)


def _combine_body(p_ref, x_ref, o_ref):
    o_ref[...] = jnp.concatenate(
        [p_ref[0] + x_ref[...], p_ref[1][:, 0:OUT - HALF]], axis=1)


def kernel(x, pos, edge_attr, We1, be1, We2, be2, Wt1, bt1, Wt2, bt2, Wo, bo,
           edge_index):
    f32 = jnp.float32
    i32 = jnp.int32
    eye4 = jnp.eye(4, dtype=f32)
    kron4 = lambda w: jnp.kron(eye4, w)
    tile4 = lambda b: jnp.tile(b, 4).reshape(1, -1)

    pad = E_PAD - E
    apad = jnp.arange(pad, dtype=i32)
    src_p = jnp.concatenate([edge_index[0], apad % N]).reshape(NCHUNK, SUB)
    dst_p = jnp.concatenate(
        [edge_index[1], N + apad % (ACC - N)]).reshape(NCHUNK, SUB)
    idx2 = jnp.stack([src_p, dst_p])

    table = pl.pallas_call(
        _table_body,
        grid=(N // TBN,),
        in_specs=[
            pl.BlockSpec((TBN, NS), lambda i: (i, 0)),
            pl.BlockSpec((TBN, 3), lambda i: (i, 0)),
        ],
        out_specs=pl.BlockSpec((TBN, ROW), lambda i: (i, 0)),
        out_shape=jax.ShapeDtypeStruct((N, ROW), f32),
    )(x, pos)
    ea4 = jnp.pad(edge_attr.reshape(E // 4, 4 * EATT),
                  ((0, (E_PAD - E) // 4), (0, 0)))

    sel = jnp.zeros((ROW, ROW), f32)
    S4 = kron4(sel.at[NS:NS + 3, :].set(1.0))
    Px4 = kron4(sel.at[NS, :].set(1.0))
    Py4 = kron4(sel.at[NS + 1, :].set(1.0))
    Pz4 = kron4(sel.at[NS + 2, :].set(1.0))

    A4 = kron4(We1[:EATT])
    B4 = kron4(We1[EATT:])
    C4 = kron4(We2)
    b1_4 = tile4(be1)
    b2_4 = tile4(be2)

    D = Wt1[:NS]
    E32 = jnp.pad(Wt1[NS:2 * NS], ((0, NS), (0, 0)))
    F32 = jnp.pad(Wt1[2 * NS:], ((0, NS), (0, 0)))
    sel01 = jnp.zeros((4, 2), f32).at[0, 0].set(1.0).at[1, 1].set(1.0)
    sel23 = jnp.zeros((4, 2), f32).at[2, 0].set(1.0).at[3, 1].set(1.0)
    Da = jnp.kron(sel01, D)
    Ea = jnp.kron(sel01, E32)
    Fa = jnp.kron(sel01, F32)
    Db = jnp.kron(sel23, D)
    Eb = jnp.kron(sel23, E32)
    Fb = jnp.kron(sel23, F32)
    bt1_2 = jnp.tile(bt1, 2).reshape(1, -1)
    G = jnp.kron(jnp.eye(2, dtype=f32),
                 jnp.pad(Wt2, ((0, 0), (0, NS))))
    bt2_4 = tile4(jnp.pad(bt2, (0, NS)))

    Wo_km = jnp.pad(
        Wo.reshape(NS, SH, OUT).transpose(1, 0, 2).reshape(NS * SH, OUT),
        ((0, 0), (0, ROW - OUT)))
    wo32 = lambda k: kron4(jnp.pad(Wo_km[k * NS:(k + 1) * NS],
                                   ((0, NS), (0, 0))))
    Wo0_4, Wox_4, Woy_4, Woz_4 = wo32(0), wo32(1), wo32(2), wo32(3)
    bo_4 = tile4(jnp.pad(bo, (0, ROW - OUT)))

    gather_kernel, scatter_kernel = _sc_kernels()
    gout = gather_kernel(table, idx2)
    g4 = gout.reshape(2, E_PAD // 4, 4 * ROW)

    grid_e = E_PAD // BE
    eb = lambda i: (i, 0)
    wb = lambda i: (0, 0)
    wspec = lambda shp: pl.BlockSpec(shp, wb)
    msg4 = pl.pallas_call(
        _edge_body,
        grid=(grid_e,),
        in_specs=[
            pl.BlockSpec((1, BE4, 4 * ROW), lambda i: (0, i, 0)),
            pl.BlockSpec((1, BE4, 4 * ROW), lambda i: (1, i, 0)),
            pl.BlockSpec((BE4, 4 * EATT), eb),
            wspec((128, 128)),
            wspec((128, 128)),
            wspec((128, 128)),
            wspec((128, 128)),
            wspec((4 * EATT, 64)),
            wspec((128, 64)),
            wspec((1, 64)),
            wspec((64, 64)),
            wspec((1, 64)),
            wspec((64, 96)),
            wspec((128, 96)),
            wspec((128, 96)),
            wspec((64, 96)),
            wspec((128, 96)),
            wspec((128, 96)),
            wspec((1, 96)),
            wspec((96, 64)),
            wspec((1, 128)),
            wspec((128, 128)),
            wspec((128, 128)),
            wspec((128, 128)),
            wspec((128, 128)),
            wspec((1, 128)),
        ],
        out_specs=pl.BlockSpec((BE4, 4 * ROW), eb),
        out_shape=jax.ShapeDtypeStruct((E_PAD // 4, 4 * ROW), f32),
    )(g4, g4, ea4, S4, Px4, Py4, Pz4, A4, B4, b1_4, C4, b2_4,
      Da, Ea, Fa, Db, Eb, Fb, bt1_2, G, bt2_4,
      Wo0_4, Wox_4, Woy_4, Woz_4, bo_4)
    msg = msg4.reshape(E_PAD, ROW)

    zeros = jnp.zeros((FLIGHT, HALF), f32)
    partials = scatter_kernel(dst_p, msg, zeros)

    grid_n = N // BN
    out = pl.pallas_call(
        _combine_body,
        grid=(grid_n,),
        in_specs=[
            pl.BlockSpec((NC, BN, HALF), lambda i: (0, i, 0)),
            pl.BlockSpec((BN, NS), lambda i: (i, 0)),
        ],
        out_specs=pl.BlockSpec((BN, OUT), lambda i: (i, 0)),
        out_shape=jax.ShapeDtypeStruct((N, OUT), f32),
    )(partials[:, :N, :], x)
    return out
